# baseline (speedup 1.0000x reference)
import functools

import jax
import jax.numpy as jnp
from jax import lax
from jax.experimental import pallas as pl
from jax.experimental.pallas import tpu as pltpu
from jax.experimental.pallas import tpu_sc as plsc

B = 512
D_MODEL = 1024
D_HIDDEN = 16384
K = 128
DEAD_THRESHOLD = 10000000.0 / 256.0

BHE = 2048
NHE = D_HIDDEN // BHE
BHD = 2048
NHD = D_HIDDEN // BHD

NW = 32
RPW = B // NW
NSLAB = RPW // 2


def _encode_body(x_ref, wenc_ref, benc_ref, bpre_ref,
                 keys_ref, mu_ref, std_ref, xs_ref):
    s = pl.program_id(0)

    @pl.when(s == 0)
    def _():
        x = x_ref[...]
        mu = jnp.mean(x, axis=-1, keepdims=True)
        xc = x - mu
        var = jnp.sum(xc * xc, axis=-1, keepdims=True) / (D_MODEL - 1)
        std = jnp.sqrt(var)
        mu_ref[...] = mu
        std_ref[...] = std
        xs_ref[...] = xc / (std + 1e-5) - bpre_ref[...]

    pre = (
        jnp.dot(xs_ref[...], wenc_ref[...], preferred_element_type=jnp.float32)
        + benc_ref[...]
    )
    keys_ref[...] = lax.bitcast_convert_type(pre, jnp.int32)


def _sc_topk_body(keys_hbm, thr_hbm, slab_a, slab_b, hist, thrbuf,
                  sem_a, sem_b):
    wid = lax.axis_index("s") * 2 + lax.axis_index("c")
    base = wid * RPW
    iota16 = lax.iota(jnp.int32, 16)
    ones16 = jnp.ones((16,), jnp.float32)

    def zero_hist(nvreg):
        def z(v, carry):
            hist[pl.ds(v * 16, 16)] = jnp.zeros((16,), jnp.float32)
            return carry
        lax.fori_loop(0, nvreg, z, jnp.int32(0))

    def suffix_find(nvreg, kneed):
        def sweep(v, carry):
            vi = (nvreg - 1) - v
            h = hist[pl.ds(vi * 16, 16)]
            total = jnp.sum(h)
            s_local = lax.rev(jnp.cumsum(lax.rev(h, (0,))), (0,))
            hist[pl.ds(vi * 16, 16)] = s_local + carry
            return carry + total
        lax.fori_loop(0, nvreg, sweep, jnp.float32(0))

        def findb(v, b):
            srow = hist[pl.ds(v * 16, 16)]
            cand = jnp.max(jnp.where(srow >= kneed, iota16 + v * 16, -1))
            return jnp.maximum(b, cand)
        b = lax.fori_loop(0, nvreg, findb, jnp.int32(-1))

        nxt = b + 1
        start = jnp.minimum((nxt // 16) * 16, (nvreg - 1) * 16)
        w = hist[pl.ds(start, 16)]
        above = jnp.sum(jnp.where(iota16 == (nxt - start), w, 0))
        above = jnp.where((b < 0) | (nxt >= nvreg * 16), 0, above)
        return b, above

    def select_row(slab, k):
        zero_hist(128)

        def p1(j, carry):
            v = slab[k, pl.ds(j * 16, 16)]
            m = v >= 0
            idx = jnp.where(m, lax.shift_right_arithmetic(v, 20), 0)
            plsc.addupdate_scatter(hist, [idx], ones16, mask=m)
            return carry
        lax.fori_loop(0, D_HIDDEN // 16, p1, jnp.int32(0))
        b1, above1 = suffix_find(128, K)
        k1 = K - above1

        zero_hist(64)

        def p2(j, carry):
            v = slab[k, pl.ds(j * 16, 16)]
            m = (v >= 0) & (lax.shift_right_arithmetic(v, 20) == b1)
            idx = jnp.where(m, lax.shift_right_arithmetic(v, 10) & 1023, 0)
            plsc.addupdate_scatter(hist, [idx], ones16, mask=m)
            return carry
        lax.fori_loop(0, D_HIDDEN // 16, p2, jnp.int32(0))
        b2, above2 = suffix_find(64, k1)
        k2 = k1 - above2

        zero_hist(64)

        def p3(j, carry):
            v = slab[k, pl.ds(j * 16, 16)]
            m = ((v >= 0) & (lax.shift_right_arithmetic(v, 20) == b1)
                 & ((lax.shift_right_arithmetic(v, 10) & 1023) == b2))
            idx = jnp.where(m, v & 1023, 0)
            plsc.addupdate_scatter(hist, [idx], ones16, mask=m)
            return carry
        lax.fori_loop(0, D_HIDDEN // 16, p3, jnp.int32(0))
        b3, _ = suffix_find(64, k2)

        thr = (b1 << 20) | (b2 << 10) | b3
        return jnp.where(b1 < 0, 0, thr)

    thrvec = jnp.zeros((16,), jnp.int32)
    pend = pltpu.async_copy(keys_hbm.at[pl.ds(base, 2)], slab_a, sem_a)
    for i in range(NSLAB):
        cur = slab_a if i % 2 == 0 else slab_b
        nxt = slab_b if i % 2 == 0 else slab_a
        nxt_sem = sem_b if i % 2 == 0 else sem_a
        pend.wait()
        if i < NSLAB - 1:
            pend = pltpu.async_copy(
                keys_hbm.at[pl.ds(base + (i + 1) * 2, 2)], nxt, nxt_sem
            )
        for k in (0, 1):
            thr = select_row(cur, k)
            thrvec = jnp.where(iota16 == (2 * i + k), thr, thrvec)
    thrbuf[...] = thrvec
    pltpu.sync_copy(thrbuf, thr_hbm.at[pl.ds(base, 16)])


def _decode_body(keys_ref, thr_ref, wdec_ref, mu_ref, std_ref, bpre_ref,
                 stats_ref, out_ref, ndead_ref, featzero_ref):
    h = pl.program_id(0)
    key = keys_ref[...]
    thr = thr_ref[...]
    lat = jnp.where(
        key >= thr, lax.bitcast_convert_type(key, jnp.float32), 0.0
    )
    part = jnp.dot(lat, wdec_ref[...], preferred_element_type=jnp.float32)

    chunk_any = jnp.max(
        (key >= jnp.maximum(thr, 1)).astype(jnp.int32), axis=0, keepdims=True
    )
    featzero_ref[:, pl.ds(h * BHD, BHD)] = 1 - chunk_any

    @pl.when(h == 0)
    def _():
        out_ref[...] = part

    @pl.when(h > 0)
    def _():
        out_ref[...] = out_ref[...] + part

    @pl.when(h == NHD - 1)
    def _():
        out_ref[...] = (
            (out_ref[...] + bpre_ref[...]) * std_ref[...] + mu_ref[...]
        )
        stats_new = stats_ref[...] * featzero_ref[...] + 1
        dead = (stats_new.astype(jnp.float32) > DEAD_THRESHOLD)
        ndead_ref[0, 0] = jnp.sum(dead.astype(jnp.int32))


_sc_topk = functools.partial(
    pl.kernel,
    out_type=jax.ShapeDtypeStruct((B,), jnp.int32),
    mesh=plsc.VectorSubcoreMesh(core_axis_name="c", subcore_axis_name="s"),
    scratch_types=[
        pltpu.VMEM((2, D_HIDDEN), jnp.int32),
        pltpu.VMEM((2, D_HIDDEN), jnp.int32),
        pltpu.VMEM((2048,), jnp.float32),
        pltpu.VMEM((16,), jnp.int32),
        pltpu.SemaphoreType.DMA,
        pltpu.SemaphoreType.DMA,
    ],
    compiler_params=pltpu.CompilerParams(needs_layout_passes=False),
)(_sc_topk_body)


@jax.jit
def kernel(x, w_enc, w_dec, b_enc, b_pre, stats_last_nonzero):
    b_enc2 = b_enc.reshape(1, D_HIDDEN)
    b_pre2 = b_pre.reshape(1, D_MODEL)
    stats2 = stats_last_nonzero.reshape(1, D_HIDDEN)

    keys, mu, std = pl.pallas_call(
        _encode_body,
        grid=(NHE,),
        in_specs=[
            pl.BlockSpec((B, D_MODEL), lambda s: (0, 0)),
            pl.BlockSpec((D_MODEL, BHE), lambda s: (0, s)),
            pl.BlockSpec((1, BHE), lambda s: (0, s)),
            pl.BlockSpec((1, D_MODEL), lambda s: (0, 0)),
        ],
        out_specs=[
            pl.BlockSpec((B, BHE), lambda s: (0, s)),
            pl.BlockSpec((B, 1), lambda s: (0, 0)),
            pl.BlockSpec((B, 1), lambda s: (0, 0)),
        ],
        out_shape=[
            jax.ShapeDtypeStruct((B, D_HIDDEN), jnp.int32),
            jax.ShapeDtypeStruct((B, 1), jnp.float32),
            jax.ShapeDtypeStruct((B, 1), jnp.float32),
        ],
        scratch_shapes=[pltpu.VMEM((B, D_MODEL), jnp.float32)],
        compiler_params=pltpu.CompilerParams(
            dimension_semantics=("arbitrary",),
        ),
    )(x, w_enc, b_enc2, b_pre2)

    thr = _sc_topk(keys)

    recons, ndead = pl.pallas_call(
        _decode_body,
        grid=(NHD,),
        in_specs=[
            pl.BlockSpec((B, BHD), lambda h: (0, h)),
            pl.BlockSpec((B, 1), lambda h: (0, 0)),
            pl.BlockSpec((BHD, D_MODEL), lambda h: (h, 0)),
            pl.BlockSpec((B, 1), lambda h: (0, 0)),
            pl.BlockSpec((B, 1), lambda h: (0, 0)),
            pl.BlockSpec((1, D_MODEL), lambda h: (0, 0)),
            pl.BlockSpec((1, D_HIDDEN), lambda h: (0, 0)),
        ],
        out_specs=[
            pl.BlockSpec((B, D_MODEL), lambda h: (0, 0)),
            pl.BlockSpec(memory_space=pltpu.SMEM),
        ],
        out_shape=[
            jax.ShapeDtypeStruct((B, D_MODEL), jnp.float32),
            jax.ShapeDtypeStruct((1, 1), jnp.int32),
        ],
        scratch_shapes=[pltpu.VMEM((1, D_HIDDEN), jnp.int32)],
        compiler_params=pltpu.CompilerParams(
            dimension_semantics=("arbitrary",),
        ),
    )(keys, thr.reshape(B, 1), w_dec, mu, std, b_pre2, stats2)

    return (recons, ndead[0, 0])
